# Initial kernel scaffold; baseline (speedup 1.0000x reference)
import jax
import jax.numpy as jnp
from jax.experimental import pallas as pl


def kernel(x, embed):
    raise NotImplementedError("write your pallas kernel here")


# speedup vs baseline: 1.4185x; 1.4185x over previous
import functools

import jax
import jax.numpy as jnp
from jax import lax
from jax.experimental import pallas as pl
from jax.experimental.pallas import tpu as pltpu
from jax.experimental.pallas import tpu_sc as plsc

_DIM = 256
_K = 1024
_TB = 512


def _argmin_body(x_ref, et_ref, idx_ref):
    et = et_ref[...]
    half_norm = 0.5 * jnp.sum(et * et, axis=0, keepdims=True)
    s = jnp.dot(x_ref[...], et, preferred_element_type=jnp.float32) - half_norm
    m = jnp.max(s, axis=-1, keepdims=True)
    iota = lax.broadcasted_iota(jnp.int32, s.shape, 1)
    idx = jnp.min(jnp.where(s == m, iota, _K), axis=-1)
    idx_ref[0, 0, :] = idx


def _nearest_index(flat, et):
    n = flat.shape[0]
    grid = n // _TB
    idx3 = pl.pallas_call(
        _argmin_body,
        grid=(grid,),
        in_specs=[
            pl.BlockSpec((_TB, _DIM), lambda i: (i, 0)),
            pl.BlockSpec((_DIM, _K), lambda i: (0, 0)),
        ],
        out_specs=pl.BlockSpec((1, 1, _TB), lambda i: (i, 0, 0)),
        out_shape=jax.ShapeDtypeStruct((grid, 1, _TB), jnp.int32),
    )(flat, et)
    return idx3.reshape(-1)


@functools.lru_cache(maxsize=None)
def _make_gather(v, d, b):
    info = plsc.get_sparse_core_info()
    nw = info.num_cores * info.num_subcores
    b_per_w = b // nw
    ch = min(b_per_w, 256)
    n_ch = b_per_w // ch
    mesh = plsc.VectorSubcoreMesh(core_axis_name="c", subcore_axis_name="s")

    @functools.partial(
        pl.kernel,
        mesh=mesh,
        out_type=jax.ShapeDtypeStruct((b, d), jnp.float32),
        scratch_types=[
            pltpu.VMEM((ch,), jnp.int32),
            pltpu.VMEM((ch, d), jnp.float32),
            pltpu.SemaphoreType.DMA,
        ],
    )
    def gather(table_hbm, idx_hbm, out_hbm, idx_v, rows_v, sem):
        wid = lax.axis_index("s") * info.num_cores + lax.axis_index("c")
        base = wid * b_per_w
        for c in range(n_ch):
            off = base + c * ch
            pltpu.sync_copy(idx_hbm.at[pl.ds(off, ch)], idx_v)
            pltpu.async_copy(table_hbm.at[idx_v], rows_v, sem).wait()
            pltpu.sync_copy(rows_v, out_hbm.at[pl.ds(off, ch)])

    return gather


def kernel(x, embed):
    b, tok, d = x.shape
    flat = x.reshape(-1, d)
    idx = _nearest_index(flat, embed.T)
    quant = _make_gather(embed.shape[0], d, flat.shape[0])(embed, idx)
    return quant.reshape(b, tok, d), idx.reshape(b, tok)


# speedup vs baseline: 1.5814x; 1.1149x over previous
import functools

import jax
import jax.numpy as jnp
from jax import lax
from jax.experimental import pallas as pl
from jax.experimental.pallas import tpu as pltpu
from jax.experimental.pallas import tpu_sc as plsc

_DIM = 256
_K = 1024
_TB = 512


def _argmin_body(x_ref, et_ref, idx_ref, hn_ref, iota_ref):
    @pl.when(pl.program_id(0) == 0)
    def _():
        et0 = et_ref[...]
        hn_ref[...] = 0.5 * jnp.sum(et0 * et0, axis=0, keepdims=True)
        iota_ref[...] = lax.broadcasted_iota(
            jnp.int32, (_TB, _K), 1).astype(jnp.float32)

    s = jnp.dot(x_ref[...], et_ref[...], preferred_element_type=jnp.float32)
    s = s - hn_ref[...]
    m = jnp.max(s, axis=-1, keepdims=True)
    idxf = jnp.min(jnp.where(s == m, iota_ref[...], float(_K)), axis=-1)
    idx_ref[0, 0, :] = idxf.astype(jnp.int32)


def _nearest_index(flat, et):
    n = flat.shape[0]
    grid = n // _TB
    idx3 = pl.pallas_call(
        _argmin_body,
        grid=(grid,),
        in_specs=[
            pl.BlockSpec((_TB, _DIM), lambda i: (i, 0)),
            pl.BlockSpec((_DIM, _K), lambda i: (0, 0)),
        ],
        out_specs=pl.BlockSpec((1, 1, _TB), lambda i: (i, 0, 0)),
        out_shape=jax.ShapeDtypeStruct((grid, 1, _TB), jnp.int32),
        scratch_shapes=[
            pltpu.VMEM((1, _K), jnp.float32),
            pltpu.VMEM((_TB, _K), jnp.float32),
        ],
    )(flat, et)
    return idx3.reshape(-1)


@functools.lru_cache(maxsize=None)
def _make_gather(v, d, b):
    info = plsc.get_sparse_core_info()
    nw = info.num_cores * info.num_subcores
    b_per_w = b // nw
    ch = min(b_per_w, 256)
    n_ch = b_per_w // ch
    mesh = plsc.VectorSubcoreMesh(core_axis_name="c", subcore_axis_name="s")

    @functools.partial(
        pl.kernel,
        mesh=mesh,
        out_type=jax.ShapeDtypeStruct((b, d), jnp.float32),
        scratch_types=[
            pltpu.VMEM((ch,), jnp.int32),
            pltpu.VMEM((ch, d), jnp.float32),
            pltpu.SemaphoreType.DMA,
        ],
    )
    def gather(table_hbm, idx_hbm, out_hbm, idx_v, rows_v, sem):
        wid = lax.axis_index("s") * info.num_cores + lax.axis_index("c")
        base = wid * b_per_w
        for c in range(n_ch):
            off = base + c * ch
            pltpu.sync_copy(idx_hbm.at[pl.ds(off, ch)], idx_v)
            pltpu.async_copy(table_hbm.at[idx_v], rows_v, sem).wait()
            pltpu.sync_copy(rows_v, out_hbm.at[pl.ds(off, ch)])

    return gather


def kernel(x, embed):
    b, tok, d = x.shape
    flat = x.reshape(-1, d)
    idx = _nearest_index(flat, embed.T)
    quant = _make_gather(embed.shape[0], d, flat.shape[0])(embed, idx)
    return quant.reshape(b, tok, d), idx.reshape(b, tok)
